# baseline (speedup 1.0000x reference)
import jax
import jax.numpy as jnp
from jax import lax
from jax.experimental import pallas as pl
from jax.experimental.pallas import tpu as pltpu
from jax.experimental.pallas import tpu_sc as plsc

NUM_USER_K = 100000
NUM_ITEM_K = 100000
HIDDEN_K = 128
BATCH_K = 1024
SCALE_K = 1.0 / 16.0

_NC = 2
_NS = 16
_NW = _NC * _NS
_B_PER_W = BATCH_K // _NW


def _sc_gather_body(table_hbm, idx_hbm, out_hbm, idx_v, rows_v, sem):
    wid = lax.axis_index("s") * _NC + lax.axis_index("c")
    base = wid * _B_PER_W
    pltpu.sync_copy(idx_hbm.at[pl.ds(base, _B_PER_W)], idx_v)
    pltpu.async_copy(table_hbm.at[idx_v], rows_v, sem).wait()
    pltpu.sync_copy(rows_v, out_hbm.at[pl.ds(base, _B_PER_W)])


def _sc_gather(user_emb, input_idx):
    k = pl.kernel(
        _sc_gather_body,
        mesh=plsc.VectorSubcoreMesh(core_axis_name="c", subcore_axis_name="s"),
        out_type=jax.ShapeDtypeStruct((BATCH_K, HIDDEN_K), jnp.float32),
        scratch_types=[
            pltpu.VMEM((_B_PER_W,), jnp.int32),
            pltpu.VMEM((_B_PER_W, HIDDEN_K), jnp.float32),
            pltpu.SemaphoreType.DMA,
        ],
    )
    return k(user_emb, input_idx)


_BN = 2048
_NSTEP = (NUM_ITEM_K + _BN - 1) // _BN


def _mm_body(a_ref, b_ref, o_ref):
    o_ref[...] = SCALE_K * lax.dot_general(
        b_ref[...], a_ref[...],
        dimension_numbers=(((1,), (1,)), ((), ())),
        preferred_element_type=jnp.float32,
    )


def _matmul_t(user_batch, item_emb):
    return pl.pallas_call(
        _mm_body,
        grid=(_NSTEP,),
        in_specs=[
            pl.BlockSpec((BATCH_K, HIDDEN_K), lambda i: (0, 0)),
            pl.BlockSpec((_BN, HIDDEN_K), lambda i: (i, 0)),
        ],
        out_specs=pl.BlockSpec((_BN, BATCH_K), lambda i: (i, 0)),
        out_shape=jax.ShapeDtypeStruct((_NSTEP * _BN, BATCH_K), jnp.float32),
    )(user_batch, item_emb)


@jax.jit
def kernel(input, input_idx, user_emb, item_emb):
    del input
    user_batch = _sc_gather(user_emb, input_idx.astype(jnp.int32))
    output_t = _matmul_t(user_batch, item_emb)
    c = jnp.zeros((BATCH_K, NUM_ITEM_K), jnp.float32)
    return (output_t, c)
